# baseline (speedup 1.0000x reference)
import functools

import jax
import jax.numpy as jnp
from jax import lax
from jax.experimental import pallas as pl
from jax.experimental.pallas import tpu as pltpu
from jax.experimental.pallas import tpu_sc as plsc

_N = 10000
_E = 320000
_F = 128
_K = 27

_NC = 2
_NS = 16
_NW = _NC * _NS
_C = 128
_NCHUNK = 2560
_E_PAD = _NCHUNK * _C
_CH0 = _NCHUNK // _NS
_ROWS_PAD = 10240
_ZROWS = 640
_DUMMY_ROW = 10008


def _sc_gather_scatter(y_flat, idx):
    mesh = plsc.VectorSubcoreMesh(core_axis_name="c", subcore_axis_name="s")

    @functools.partial(
        pl.kernel,
        mesh=mesh,
        out_type=jax.ShapeDtypeStruct((1, _ROWS_PAD, _F), jnp.float32),
        scratch_types=[
            pltpu.VMEM((2, 2, _C), jnp.int32),
            pltpu.VMEM((2, _C, _F), jnp.float32),
            pltpu.VMEM_SHARED((_ROWS_PAD, _F), jnp.float32),
            pltpu.SemaphoreType.DMA,
            pltpu.SemaphoreType.DMA,
        ],
    )
    def k(y_hbm, idx_hbm, out_hbm, idx_v, rows_v, acc, sem0, sem1):
        cid = lax.axis_index("c")
        sid = lax.axis_index("s")

        @pl.when(cid == 0)
        def _core0():
            def zrow(i, carry):
                for j in range(_F // 16):
                    rows_v[0, i, pl.ds(j * 16, 16)] = jnp.zeros(
                        (16,), jnp.float32)
                return carry

            lax.fori_loop(0, _C, zrow, 0)
            base_z = sid * _ZROWS
            for j in range(_ZROWS // _C):
                pltpu.sync_copy(rows_v.at[0],
                                acc.at[pl.ds(base_z + j * _C, _C)])
            plsc.subcore_barrier()

            sems = (sem0, sem1)
            base_c = sid * _CH0

            def load_fire(ci, b):
                pltpu.sync_copy(idx_hbm.at[base_c + ci], idx_v.at[b])
                pltpu.async_copy(y_hbm.at[idx_v.at[b, 0]], rows_v.at[b],
                                 sems[b])

            def drain(b):
                pltpu.make_async_copy(y_hbm.at[pl.ds(0, _C)], rows_v.at[b],
                                      sems[b]).wait()

            def scatter(b):
                pltpu.sync_copy(rows_v.at[b], acc.at[idx_v.at[b, 1]],
                                add=True)

            load_fire(0, 0)

            def step(j, carry):
                ci = j * 2
                load_fire(ci + 1, 1)
                drain(0)
                scatter(0)

                @pl.when(j < _CH0 // 2 - 1)
                def _refire():
                    load_fire(ci + 2, 0)

                drain(1)
                scatter(1)
                return carry

            lax.fori_loop(0, _CH0 // 2, step, 0)
            plsc.subcore_barrier()

            pltpu.sync_copy(acc.at[pl.ds(base_z, _ZROWS)],
                            out_hbm.at[0, pl.ds(base_z, _ZROWS)])

    return k(y_flat, idx)


def _tc_einsum(feat, W):
    def body(f_ref, w_ref, y_ref):
        y_ref[0] = jnp.dot(f_ref[...], w_ref[0], preferred_element_type=jnp.float32)

    return pl.pallas_call(
        body,
        grid=(_K,),
        in_specs=[
            pl.BlockSpec((_N, _F), lambda k: (0, 0)),
            pl.BlockSpec((1, _F, _F), lambda k: (k, 0, 0)),
        ],
        out_specs=pl.BlockSpec((1, _N, _F), lambda k: (k, 0, 0)),
        out_shape=jax.ShapeDtypeStruct((_K, _N, _F), jnp.float32),
    )(feat, W)


def _tc_bn_silu(p, gamma, beta):
    def body(p_ref, g_ref, b_ref, o_ref):
        h = p_ref[0, :_N]
        mu = jnp.mean(h, axis=0, keepdims=True)
        var = jnp.mean(jnp.square(h - mu), axis=0, keepdims=True)
        hn = (h - mu) * lax.rsqrt(var + 1e-5) * g_ref[...] + b_ref[...]
        o_ref[...] = hn * jax.nn.sigmoid(hn)

    return pl.pallas_call(
        body,
        out_shape=jax.ShapeDtypeStruct((_N, _F), jnp.float32),
    )(p, gamma.reshape(1, _F), beta.reshape(1, _F))


def _tc_final(p, z, Wp, bp, gamma_p, beta_p):
    def body(p_ref, z_ref, w_ref, bp_ref, g_ref, b_ref, o_ref):
        h = p_ref[0, :_N]
        zp = jnp.dot(z_ref[...], w_ref[...], preferred_element_type=jnp.float32)
        zp = zp + bp_ref[...]
        mu = jnp.mean(zp, axis=0, keepdims=True)
        var = jnp.mean(jnp.square(zp - mu), axis=0, keepdims=True)
        zpn = (zp - mu) * lax.rsqrt(var + 1e-5) * g_ref[...] + b_ref[...]
        zpn = jnp.maximum(zpn, 0.0)
        o_ref[...] = h + zpn

    return pl.pallas_call(
        body,
        out_shape=jax.ShapeDtypeStruct((_N, _F), jnp.float32),
    )(p, z, Wp, bp.reshape(1, _F), gamma_p.reshape(1, _F),
      beta_p.reshape(1, _F))


def kernel(x, z, edge_index, kernel_offset, W1, gamma1, beta1, W2, Wp, bp, gamma_p, beta_p):
    src = edge_index[0]
    dst = edge_index[1]
    flat_src = kernel_offset * _N + src
    pad = _E_PAD - _E
    flat_src = jnp.concatenate(
        [flat_src, jnp.zeros((pad,), jnp.int32)]).reshape(_E_PAD // _C, _C)
    dst_p = jnp.concatenate(
        [dst, jnp.full((pad,), _DUMMY_ROW, jnp.int32)]).reshape(_E_PAD // _C, _C)
    idx = jnp.stack([flat_src, dst_p], axis=1)

    y1 = _tc_einsum(x, W1).reshape(_K * _N, _F)
    p1 = _sc_gather_scatter(y1, idx)
    h1 = _tc_bn_silu(p1, gamma1, beta1)
    y2 = _tc_einsum(h1, W2).reshape(_K * _N, _F)
    p2 = _sc_gather_scatter(y2, idx)
    out = _tc_final(p2, z, Wp, bp, gamma_p, beta_p)
    return (out, out)


# speedup vs baseline: 1.2299x; 1.2299x over previous
import functools

import jax
import jax.numpy as jnp
from jax import lax
from jax.experimental import pallas as pl
from jax.experimental.pallas import tpu as pltpu
from jax.experimental.pallas import tpu_sc as plsc

_N = 10000
_E = 320000
_F = 128
_K = 27

_NC = 2
_NS = 16
_NW = _NC * _NS
_C = 128
_NCHUNK = 2560
_E_PAD = _NCHUNK * _C
_CH0 = 120
_CH1 = 40
_ROWS_PAD = 10240
_ZROWS = 640
_DUMMY_ROW = 10008


def _sc_gather_scatter(y_flat, idx):
    mesh = plsc.VectorSubcoreMesh(core_axis_name="c", subcore_axis_name="s")

    @functools.partial(
        pl.kernel,
        mesh=mesh,
        out_type=jax.ShapeDtypeStruct((_NC, _ROWS_PAD, _F), jnp.float32),
        scratch_types=[
            pltpu.VMEM((2, 2, _C), jnp.int32),
            pltpu.VMEM((2, _C, _F), jnp.float32),
            pltpu.VMEM_SHARED((_ROWS_PAD, _F), jnp.float32),
            pltpu.SemaphoreType.DMA,
            pltpu.SemaphoreType.DMA,
        ],
    )
    def k(y_hbm, idx_hbm, out_hbm, idx_v, rows_v, acc, sem0, sem1):
        cid = lax.axis_index("c")
        sid = lax.axis_index("s")

        def zrow(i, carry):
            for j in range(_F // 16):
                rows_v[0, i, pl.ds(j * 16, 16)] = jnp.zeros((16,), jnp.float32)
            return carry

        lax.fori_loop(0, _C, zrow, 0)
        base_z = sid * _ZROWS
        for j in range(_ZROWS // _C):
            pltpu.sync_copy(rows_v.at[0], acc.at[pl.ds(base_z + j * _C, _C)])
        plsc.subcore_barrier()

        sems = (sem0, sem1)

        @pl.when(cid == 0)
        def _fast_core():
            base_c = sid * _CH0

            def load_fire(ci, b):
                pltpu.sync_copy(idx_hbm.at[base_c + ci], idx_v.at[b])
                pltpu.async_copy(y_hbm.at[idx_v.at[b, 0]], rows_v.at[b],
                                 sems[b])

            def drain(b):
                pltpu.make_async_copy(y_hbm.at[pl.ds(0, _C)], rows_v.at[b],
                                      sems[b]).wait()

            def scatter(b):
                pltpu.sync_copy(rows_v.at[b], acc.at[idx_v.at[b, 1]],
                                add=True)

            load_fire(0, 0)

            def step(j, carry):
                ci = j * 2
                load_fire(ci + 1, 1)
                drain(0)
                scatter(0)

                @pl.when(j < _CH0 // 2 - 1)
                def _refire():
                    load_fire(ci + 2, 0)

                drain(1)
                scatter(1)
                return carry

            lax.fori_loop(0, _CH0 // 2, step, 0)

        @pl.when(cid == 1)
        def _slow_core():
            base_c = _NS * _CH0 + sid * _CH1

            def step(i, carry):
                pltpu.sync_copy(idx_hbm.at[base_c + i], idx_v.at[0])
                pltpu.async_copy(y_hbm.at[idx_v.at[0, 0]], rows_v.at[0],
                                 sem0).wait()
                pltpu.sync_copy(rows_v.at[0], acc.at[idx_v.at[0, 1]],
                                add=True)
                return carry

            lax.fori_loop(0, _CH1, step, 0)

        plsc.subcore_barrier()

        pltpu.sync_copy(acc.at[pl.ds(base_z, _ZROWS)],
                        out_hbm.at[cid, pl.ds(base_z, _ZROWS)])

    return k(y_flat, idx)


def _tc_einsum(feat, W):
    def body(f_ref, w_ref, y_ref):
        y_ref[0] = jnp.dot(f_ref[...], w_ref[0], preferred_element_type=jnp.float32)

    return pl.pallas_call(
        body,
        grid=(_K,),
        in_specs=[
            pl.BlockSpec((_N, _F), lambda k: (0, 0)),
            pl.BlockSpec((1, _F, _F), lambda k: (k, 0, 0)),
        ],
        out_specs=pl.BlockSpec((1, _N, _F), lambda k: (k, 0, 0)),
        out_shape=jax.ShapeDtypeStruct((_K, _N, _F), jnp.float32),
    )(feat, W)


def _tc_bn_silu(p, gamma, beta):
    def body(p_ref, g_ref, b_ref, o_ref):
        h = p_ref[0, :_N] + p_ref[1, :_N]
        mu = jnp.mean(h, axis=0, keepdims=True)
        var = jnp.mean(jnp.square(h - mu), axis=0, keepdims=True)
        hn = (h - mu) * lax.rsqrt(var + 1e-5) * g_ref[...] + b_ref[...]
        o_ref[...] = hn * jax.nn.sigmoid(hn)

    return pl.pallas_call(
        body,
        out_shape=jax.ShapeDtypeStruct((_N, _F), jnp.float32),
    )(p, gamma.reshape(1, _F), beta.reshape(1, _F))


def _tc_final(p, z, Wp, bp, gamma_p, beta_p):
    def body(p_ref, z_ref, w_ref, bp_ref, g_ref, b_ref, o_ref):
        h = p_ref[0, :_N] + p_ref[1, :_N]
        zp = jnp.dot(z_ref[...], w_ref[...], preferred_element_type=jnp.float32)
        zp = zp + bp_ref[...]
        mu = jnp.mean(zp, axis=0, keepdims=True)
        var = jnp.mean(jnp.square(zp - mu), axis=0, keepdims=True)
        zpn = (zp - mu) * lax.rsqrt(var + 1e-5) * g_ref[...] + b_ref[...]
        zpn = jnp.maximum(zpn, 0.0)
        o_ref[...] = h + zpn

    return pl.pallas_call(
        body,
        out_shape=jax.ShapeDtypeStruct((_N, _F), jnp.float32),
    )(p, z, Wp, bp.reshape(1, _F), gamma_p.reshape(1, _F),
      beta_p.reshape(1, _F))


def kernel(x, z, edge_index, kernel_offset, W1, gamma1, beta1, W2, Wp, bp, gamma_p, beta_p):
    src = edge_index[0]
    dst = edge_index[1]
    flat_src = kernel_offset * _N + src
    pad = _E_PAD - _E
    flat_src = jnp.concatenate(
        [flat_src, jnp.zeros((pad,), jnp.int32)]).reshape(_NCHUNK, _C)
    dst_p = jnp.concatenate(
        [dst, jnp.full((pad,), _DUMMY_ROW, jnp.int32)]).reshape(_NCHUNK, _C)
    idx = jnp.stack([flat_src, dst_p], axis=1)

    y1 = _tc_einsum(x, W1).reshape(_K * _N, _F)
    p1 = _sc_gather_scatter(y1, idx)
    h1 = _tc_bn_silu(p1, gamma1, beta1)
    y2 = _tc_einsum(h1, W2).reshape(_K * _N, _F)
    p2 = _sc_gather_scatter(y2, idx)
    out = _tc_final(p2, z, Wp, bp, gamma_p, beta_p)
    return (out, out)


# speedup vs baseline: 1.3851x; 1.1262x over previous
import functools

import jax
import jax.numpy as jnp
from jax import lax
from jax.experimental import pallas as pl
from jax.experimental.pallas import tpu as pltpu
from jax.experimental.pallas import tpu_sc as plsc

_N = 10000
_E = 320000
_F = 128
_K = 27

_NC = 2
_NS = 16
_NW = _NC * _NS
_C = 128
_CHUNKS = 79
_PER_W = _C * _CHUNKS
_E_PAD = _PER_W * _NW
_ROWS_PAD = 10240
_ZROWS = 640
_DUMMY_ROW = 10008


def _sc_gather_scatter(y_flat, src_flat, dst):
    mesh = plsc.VectorSubcoreMesh(core_axis_name="c", subcore_axis_name="s")

    @functools.partial(
        pl.kernel,
        mesh=mesh,
        out_type=jax.ShapeDtypeStruct((_NC, _ROWS_PAD, _F), jnp.float32),
        scratch_types=[
            pltpu.VMEM((_C,), jnp.int32),
            pltpu.VMEM((_C,), jnp.int32),
            pltpu.VMEM((_C, _F), jnp.float32),
            pltpu.VMEM((_C, _F), jnp.float32),
            pltpu.VMEM_SHARED((_ROWS_PAD, _F), jnp.float32),
            pltpu.SemaphoreType.DMA,
        ],
    )
    def k(y_hbm, src_hbm, dst_hbm, out_hbm, src_v, dst_v, rows_v, zero_v, acc, sem):
        cid = lax.axis_index("c")
        sid = lax.axis_index("s")
        wid = sid * _NC + cid

        def zrow(i, carry):
            for j in range(_F // 16):
                zero_v[i, pl.ds(j * 16, 16)] = jnp.zeros((16,), jnp.float32)
            return carry

        lax.fori_loop(0, _C, zrow, 0)

        base_z = sid * _ZROWS
        for j in range(_ZROWS // _C):
            pltpu.sync_copy(zero_v, acc.at[pl.ds(base_z + j * _C, _C)])
        plsc.subcore_barrier()

        base_e = wid * _PER_W

        def step(i, carry):
            off = pl.multiple_of(base_e + i * _C, _C)
            pltpu.sync_copy(src_hbm.at[pl.ds(off, _C)], src_v)
            pltpu.sync_copy(dst_hbm.at[pl.ds(off, _C)], dst_v)
            pltpu.async_copy(y_hbm.at[src_v], rows_v, sem).wait()
            pltpu.sync_copy(rows_v, acc.at[dst_v], add=True)
            return carry

        lax.fori_loop(0, _CHUNKS, step, 0)
        plsc.subcore_barrier()

        pltpu.sync_copy(acc.at[pl.ds(base_z, _ZROWS)],
                        out_hbm.at[cid, pl.ds(base_z, _ZROWS)])

    return k(y_flat, src_flat, dst)


def _tc_einsum(feat, W):
    def body(f_ref, w_ref, y_ref):
        y_ref[0] = jnp.dot(f_ref[...], w_ref[0], preferred_element_type=jnp.float32)

    return pl.pallas_call(
        body,
        grid=(_K,),
        in_specs=[
            pl.BlockSpec((_N, _F), lambda k: (0, 0)),
            pl.BlockSpec((1, _F, _F), lambda k: (k, 0, 0)),
        ],
        out_specs=pl.BlockSpec((1, _N, _F), lambda k: (k, 0, 0)),
        out_shape=jax.ShapeDtypeStruct((_K, _N, _F), jnp.float32),
    )(feat, W)


def _tc_bn_silu(p, gamma, beta):
    def body(p_ref, g_ref, b_ref, o_ref):
        h = p_ref[0, :_N] + p_ref[1, :_N]
        mu = jnp.mean(h, axis=0, keepdims=True)
        var = jnp.mean(jnp.square(h - mu), axis=0, keepdims=True)
        hn = (h - mu) * lax.rsqrt(var + 1e-5) * g_ref[...] + b_ref[...]
        o_ref[...] = hn * jax.nn.sigmoid(hn)

    return pl.pallas_call(
        body,
        out_shape=jax.ShapeDtypeStruct((_N, _F), jnp.float32),
    )(p, gamma.reshape(1, _F), beta.reshape(1, _F))


def _tc_final(p, z, Wp, bp, gamma_p, beta_p):
    def body(p_ref, z_ref, w_ref, bp_ref, g_ref, b_ref, o_ref):
        h = p_ref[0, :_N] + p_ref[1, :_N]
        zp = jnp.dot(z_ref[...], w_ref[...], preferred_element_type=jnp.float32)
        zp = zp + bp_ref[...]
        mu = jnp.mean(zp, axis=0, keepdims=True)
        var = jnp.mean(jnp.square(zp - mu), axis=0, keepdims=True)
        zpn = (zp - mu) * lax.rsqrt(var + 1e-5) * g_ref[...] + b_ref[...]
        zpn = jnp.maximum(zpn, 0.0)
        o_ref[...] = h + zpn

    return pl.pallas_call(
        body,
        out_shape=jax.ShapeDtypeStruct((_N, _F), jnp.float32),
    )(p, z, Wp, bp.reshape(1, _F), gamma_p.reshape(1, _F),
      beta_p.reshape(1, _F))


def kernel(x, z, edge_index, kernel_offset, W1, gamma1, beta1, W2, Wp, bp, gamma_p, beta_p):
    src = edge_index[0]
    dst = edge_index[1]
    flat_src = kernel_offset * _N + src
    pad = _E_PAD - _E
    flat_src = jnp.concatenate([flat_src, jnp.zeros((pad,), jnp.int32)])
    dst_p = jnp.concatenate([dst, jnp.full((pad,), _DUMMY_ROW, jnp.int32)])

    y1 = _tc_einsum(x, W1).reshape(_K * _N, _F)
    p1 = _sc_gather_scatter(y1, flat_src, dst_p)
    h1 = _tc_bn_silu(p1, gamma1, beta1)
    y2 = _tc_einsum(h1, W2).reshape(_K * _N, _F)
    p2 = _sc_gather_scatter(y2, flat_src, dst_p)
    out = _tc_final(p2, z, Wp, bp, gamma_p, beta_p)
    return (out, out)
